# Initial kernel scaffold; baseline (speedup 1.0000x reference)
import jax
import jax.numpy as jnp
from jax.experimental import pallas as pl


def kernel(tokens, values, covariates, cat_table, num_table, W_static, b_static):
    raise NotImplementedError("write your pallas kernel here")


# speedup vs baseline: 1.5131x; 1.5131x over previous
import jax
import jax.numpy as jnp
from jax import lax
from jax.experimental import pallas as pl
from jax.experimental.pallas import tpu as pltpu
from jax.experimental.pallas import tpu_sc as plsc

VOCAB = 1000000
D = 32
NS = 16
B, L = 4096, 200
NW = 32
BPW = B // NW
LANES = 16
LPAD = 208
NGRP = LPAD // LANES


def _sc_body(tokens_hbm, values_hbm, cov_hbm, cat_hbm, num_hbm, w_hbm, bias_hbm,
             out_hbm,
             idx_v, vals_v, cov_v, w_v, bias_v, static_v, cat_v, num_v,
             gsem):
    cid = lax.axis_index("c")
    sid = lax.axis_index("s")
    wid = sid * 2 + cid
    b0 = wid * BPW

    pltpu.sync_copy(cov_hbm.at[pl.ds(b0, BPW)], cov_v)
    pltpu.sync_copy(w_hbm, w_v)
    pltpu.sync_copy(bias_hbm, bias_v)

    bias0 = bias_v[pl.ds(0, LANES)]
    bias1 = bias_v[pl.ds(LANES, LANES)]

    def _proj_row(r, _):
        cr = cov_v[r, pl.ds(0, NS)]
        a0, a1 = bias0, bias1
        for k in range(NS):
            ck = jnp.full((LANES,), cr[k], dtype=jnp.float32)
            a0 = a0 + ck * w_v[k, pl.ds(0, LANES)]
            a1 = a1 + ck * w_v[k, pl.ds(LANES, LANES)]
        static_v[r, pl.ds(0, LANES)] = a0
        static_v[r, pl.ds(LANES, LANES)] = a1
        return _
    lax.fori_loop(0, BPW, _proj_row, None)

    def _row(i, _):
        b = b0 + i
        pltpu.sync_copy(tokens_hbm.at[pl.ds(b * L, L)], idx_v)
        pltpu.sync_copy(values_hbm.at[pl.ds(b * L, L)], vals_v.at[pl.ds(0, L)])
        cps = []
        for (lo, n) in ((0, 128), (128, 72)):
            cps.append(pltpu.async_copy(
                cat_hbm.at[idx_v.at[pl.ds(lo, n)]], cat_v.at[pl.ds(lo, n)],
                gsem))
            cps.append(pltpu.async_copy(
                num_hbm.at[idx_v.at[pl.ds(lo, n)]], num_v.at[pl.ds(lo, n)],
                gsem))
        for cp in cps:
            cp.wait()

        st0 = static_v[i, pl.ds(0, LANES)]
        st1 = static_v[i, pl.ds(LANES, LANES)]

        def _grp(g, _):
            vblk = vals_v[pl.ds(g * LANES, LANES)]
            vblk = jnp.where(vblk == vblk, vblk, jnp.float32(0.0))
            for c in range(LANES):
                row = g * LANES + c
                vb = jnp.full((LANES,), vblk[c], dtype=jnp.float32)
                cat_v[row, pl.ds(0, LANES)] = (
                    cat_v[row, pl.ds(0, LANES)]
                    + vb * num_v[row, pl.ds(0, LANES)] + st0)
                cat_v[row, pl.ds(LANES, LANES)] = (
                    cat_v[row, pl.ds(LANES, LANES)]
                    + vb * num_v[row, pl.ds(LANES, LANES)] + st1)
            return _
        lax.fori_loop(0, NGRP, _grp, None)

        pltpu.sync_copy(cat_v.at[pl.ds(0, L)], out_hbm.at[b])
        return _
    lax.fori_loop(0, BPW, _row, None)


@jax.jit
def _embed(tokens, values, covariates, cat_table, num_table, W_static, b_static):
    mesh = plsc.VectorSubcoreMesh(core_axis_name="c", subcore_axis_name="s",
                                  num_cores=2, num_subcores=16)
    f = pl.kernel(
        _sc_body,
        out_type=jax.ShapeDtypeStruct((B, L, D), jnp.float32),
        mesh=mesh,
        scratch_types=[
            pltpu.VMEM((L,), jnp.int32),
            pltpu.VMEM((LPAD,), jnp.float32),
            pltpu.VMEM((BPW, NS), jnp.float32),
            pltpu.VMEM((NS, D), jnp.float32),
            pltpu.VMEM((D,), jnp.float32),
            pltpu.VMEM((BPW, D), jnp.float32),
            pltpu.VMEM((LPAD, D), jnp.float32),
            pltpu.VMEM((LPAD, D), jnp.float32),
            pltpu.SemaphoreType.DMA,
        ],
        compiler_params=pltpu.CompilerParams(use_tc_tiling_on_sc=False),
    )
    return f(tokens.reshape(B * L), values.reshape(B * L), covariates,
             cat_table, num_table, W_static, b_static)


def kernel(tokens, values, covariates, cat_table, num_table, W_static, b_static):
    return _embed(tokens.astype(jnp.int32), values, covariates,
                  cat_table, num_table, W_static, b_static)
